# Initial kernel scaffold; baseline (speedup 1.0000x reference)
import jax
import jax.numpy as jnp
from jax.experimental import pallas as pl


def kernel(x, params, edge_index, batch, target):
    raise NotImplementedError("write your pallas kernel here")


# speedup vs baseline: 1.2180x; 1.2180x over previous
import jax
import jax.numpy as jnp
import numpy as np
from jax.experimental import pallas as pl
from jax.experimental.pallas import tpu as pltpu

_AVG_LOG = float(np.log(17.0))
_NN = 50000
_NB = 128


def _tail_body(xc, w1, b1, w2, b2, w3, b3, out):
    h = jnp.maximum(xc[...] @ w1[...] + b1[...], 0.0)
    h = jnp.maximum(h @ w2[...] + b2[...], 0.0)
    out[...] = h @ w3[...] + b3[...]


def _tail(xc, p):
    return pl.pallas_call(
        _tail_body,
        out_shape=jax.ShapeDtypeStruct((_NB, 1), jnp.float32),
    )(xc, p['fc1_W'], p['fc1_b'][None, :], p['fc2_W'], p['fc2_b'][None, :],
      p['out_W'], p['out_b'][None, :])


def _pna_layer(h, src, dst, deg, degc, dlog, p, n):
    fi = h.shape[1]
    preW = p['c%d_pre_W' % n]
    a = h @ preW[:fi]
    b = h @ preW[fi:] + p['c%d_pre_b' % n]
    bs = b[src]
    S1 = jax.ops.segment_sum(bs, dst, num_segments=_NN)
    S2 = jax.ops.segment_sum(bs * bs, dst, num_segments=_NN)
    MX = jax.ops.segment_max(bs, dst, num_segments=_NN)
    MN = jax.ops.segment_min(bs, dst, num_segments=_NN)
    degcol = deg[:, None]
    mean = (degcol * a + S1) / degc
    sq = (degcol * (a * a) + 2.0 * a * S1 + S2) / degc
    std = jnp.sqrt(jax.nn.relu(sq - mean * mean) + 1e-5)
    has = (deg > 0.0)[:, None]
    mx = jnp.where(has, a + MX, 0.0)
    mn = jnp.where(has, a + MN, 0.0)
    aggr = jnp.concatenate([mean, mx, mn, std], axis=-1)
    out = jnp.concatenate(
        [h, aggr, aggr * (dlog / _AVG_LOG), aggr * (_AVG_LOG / dlog)], axis=-1)
    out = out @ p['c%d_post_W' % n] + p['c%d_post_b' % n]
    out = out @ p['c%d_lin_W' % n] + p['c%d_lin_b' % n]
    out = (out - p['bn%d_rm' % n]) / jnp.sqrt(p['bn%d_rv' % n] + 1e-5) \
        * p['bn%d_g' % n] + p['bn%d_b' % n]
    return jax.nn.relu(out)


def kernel(x, params, edge_index, batch, target):
    p = params
    src, dst = edge_index[0], edge_index[1]
    deg = jax.ops.segment_sum(jnp.ones((src.shape[0],), jnp.float32), dst,
                              num_segments=_NN)
    degc = jnp.clip(deg, 1.0)[:, None]
    dlog = jnp.log(jnp.clip(deg, 1.0) + 1.0)[:, None]

    h = x
    for n in (1, 2, 3):
        h = _pna_layer(h, src, dst, deg, degc, dlog, p, n)

    sums = jax.ops.segment_sum(h, batch, num_segments=_NB)
    cnt = jnp.clip(jax.ops.segment_sum(jnp.ones((_NN,), jnp.float32), batch,
                                       num_segments=_NB), 1.0)[:, None]
    xg = jax.nn.relu((sums / cnt) @ p['fc1_xd_W'] + p['fc1_xd_b'])

    emb = p['emb'][target]
    et = jnp.transpose(emb, (0, 2, 1))
    conv = jax.lax.conv_general_dilated(
        et, p['cxt_W'], (1,), 'VALID',
        dimension_numbers=('NCH', 'OIH', 'NCH')) + p['cxt_b'][None, :, None]
    xt = conv.reshape(_NB, 32 * 78) @ p['fc1_xt_W'] + p['fc1_xt_b']

    xc = jnp.concatenate([xg, xt], axis=1)
    return _tail(xc, p)


# speedup vs baseline: 7.7362x; 6.3513x over previous
import functools

import jax
import jax.numpy as jnp
import numpy as np
from jax import lax
from jax.experimental import pallas as pl
from jax.experimental.pallas import tpu as pltpu
from jax.experimental.pallas import tpu_sc as plsc

_AVG_LOG = float(np.log(17.0))
_NN = 50000
_NB = 128
_NP = 50176
_NPT = 1568
_EH = 400000
_ET = 25000
_W = 384
_SW = 2048
_SWT = 432
_NPS = 3136
_FLUSH = 32
_BIG = 3.0e38
_D = 128


def _shift_down(v, s, iota):
    return v[jnp.maximum(iota - s, 0)]


def _sort_body(edge, zeros, src_out, off_out,
               win_d, win_s, idx_b, pos_b, ones_b,
               win_dt, win_st, idx_bt, pos_bt, ones_bt,
               g16, ones16, scan_buf, off_stage, tot_w, tot_v,
               hist, totals_sp, sem):
    c = lax.axis_index("c")
    t = lax.axis_index("s")
    iota = lax.iota(jnp.int32, 16)
    ebase = c * _EH + t * _ET
    srow = 0
    drow = _EH * 2 + 16

    z0 = pl.multiple_of(t * 50176, 8)
    pltpu.sync_copy(zeros.at[pl.ds(z0, 50176)], hist.at[pl.ds(z0, 50176)])

    @pl.when(t == 0)
    def _():
        pltpu.sync_copy(zeros.at[pl.ds(0, 128)],
                        hist.at[pl.ds(802816, 128)])

    def fill_ones(ref, n):
        def fb(i, carry):
            ref[pl.ds(i * 16, 16)] = jnp.ones((16,), jnp.int32)
            return carry
        lax.fori_loop(0, n // 16, fb, 0)

    fill_ones(ones_b, _SW)
    fill_ones(ones_bt, _SWT)
    ones16[pl.ds(0, 16)] = jnp.ones((16,), jnp.int32)
    plsc.subcore_barrier()

    def hist_window(base_e, nw, nvalid, wd, ib, ob):
        pltpu.sync_copy(edge.at[pl.ds(drow + base_e, nw)], wd)

        def ib_body(i, carry):
            d16 = wd[pl.ds(i * 16, 16)]
            valid = (i * 16 + iota) < nvalid
            ib[pl.ds(i * 16, 16)] = jnp.where(valid, d16 * 16 + t,
                                              802816 + t)
            return carry
        lax.fori_loop(0, nw // 16, ib_body, 0)
        pltpu.sync_copy(ob, hist.at[ib], add=True)

    for w in range(12):
        hist_window(pl.multiple_of(ebase + w * _SW, 8), _SW, _SW,
                    win_d, idx_b, ones_b)
    hist_window(pl.multiple_of(ebase + 12 * _SW, 8), _SWT, _ET - 12 * _SW,
                win_dt, idx_bt, ones_bt)
    plsc.subcore_barrier()

    nscan0 = pl.multiple_of(t * _NPS * 16, 8)
    pltpu.sync_copy(hist.at[pl.ds(nscan0, _NPS * 16)], scan_buf)

    def group_body(g, carry):
        off_acc = jnp.zeros((16,), jnp.int32)
        for j in range(16):
            b0 = pl.multiple_of((g * 16 + j) * 16, 8)
            v = scan_buf[pl.ds(b0, 16)]
            incl = v
            for s in (1, 2, 4, 8):
                incl = incl + jnp.where(iota >= s,
                                        _shift_down(incl, s, iota), 0)
            total = incl[15]
            scan_buf[pl.ds(b0, 16)] = (incl - v) + carry
            off_acc = jnp.where(iota == j, carry, off_acc)
            carry = carry + total
        off_stage[pl.ds(pl.multiple_of(g * 16, 8), 16)] = off_acc
        return carry

    range_total = lax.fori_loop(0, _NPS // 16, group_body, jnp.int32(0))
    for i in range(4):
        tot_w[pl.ds(i * 16, 16)] = jnp.zeros((16,), jnp.int32) + range_total
    pltpu.sync_copy(tot_w, totals_sp.at[pl.ds(pl.multiple_of(t * 64, 8), 64)])
    plsc.subcore_barrier()

    my_base = jnp.int32(0)
    pltpu.sync_copy(totals_sp, tot_v)
    for t2 in range(16):
        v = tot_v[pl.ds(t2 * 64, 16)]
        my_base = my_base + jnp.where(t2 < t, v[0], 0)

    def add_base(i, carry):
        b0 = pl.multiple_of(i * 16, 8)
        scan_buf[pl.ds(b0, 16)] = scan_buf[pl.ds(b0, 16)] + my_base
        return carry
    lax.fori_loop(0, _NPS, add_base, 0)

    def add_base_off(i, carry):
        b0 = pl.multiple_of(i * 16, 8)
        off_stage[pl.ds(b0, 16)] = off_stage[pl.ds(b0, 16)] + my_base
        return carry
    lax.fori_loop(0, _NPS // 16, add_base_off, 0)

    pltpu.sync_copy(scan_buf, hist.at[pl.ds(nscan0, _NPS * 16)])
    @pl.when(t == 15)
    def _():
        def pb(i, carry):
            off_stage[pl.ds(pl.multiple_of(_NPS + i * 16, 8), 16)] = (
                jnp.zeros((16,), jnp.int32) + jnp.int32(_EH))
            return carry
        lax.fori_loop(0, 2, pb, 0)

    @pl.when(t < 15)
    def _():
        pltpu.sync_copy(
            off_stage.at[pl.ds(0, _NPS)],
            off_out.at[pl.ds(pl.multiple_of(c * (_NP + 32) + t * _NPS, 8),
                             _NPS)])

    @pl.when(t == 15)
    def _():
        pltpu.sync_copy(
            off_stage,
            off_out.at[pl.ds(pl.multiple_of(c * (_NP + 32) + 15 * _NPS, 8),
                             _NPS + 32)])
    plsc.subcore_barrier()

    def perm_window(base_e, nw, nvalid, wd, ws, pb):
        pltpu.sync_copy(edge.at[pl.ds(drow + base_e, nw)], wd)
        pltpu.sync_copy(edge.at[pl.ds(srow + base_e, nw)], ws)

        def vb(i, carry):
            d16 = wd[pl.ds(i * 16, 16)]
            valid = (i * 16 + iota) < nvalid
            idx16 = jnp.where(valid, d16 * 16 + t, 802816 + t)
            pltpu.sync_copy(hist.at[idx16], g16)
            b16 = g16[pl.ds(0, 16)]
            rank = jnp.zeros((16,), jnp.int32)
            for s in range(1, 16):
                dn = _shift_down(d16, s, iota)
                rank = rank + jnp.where((iota >= s) & (dn == d16), 1, 0)
            pb[pl.ds(i * 16, 16)] = (c * (_EH + 512)
                                     + jnp.where(valid, b16 + rank,
                                                 _EH + 16 + iota))
            pltpu.sync_copy(ones16, hist.at[idx16], add=True)
            return carry
        lax.fori_loop(0, nw // 16, vb, 0)
        pltpu.sync_copy(ws, src_out.at[pb])

    for w in range(12):
        perm_window(pl.multiple_of(ebase + w * _SW, 8), _SW, _SW,
                    win_d, win_s, pos_b)
    perm_window(pl.multiple_of(ebase + 12 * _SW, 8), _SWT, _ET - 12 * _SW,
                win_dt, win_st, pos_bt)
    plsc.subcore_barrier()

    @pl.when(t == 0)
    def _():
        pltpu.sync_copy(
            ones_b.at[pl.ds(0, 512)],
            src_out.at[pl.ds(pl.multiple_of(c * (_EH + 512) + _EH, 8), 512)])


def _sc_sort(edge_index, zeros):
    mesh = plsc.VectorSubcoreMesh(core_axis_name="c", subcore_axis_name="s")
    fn = pl.kernel(
        _sort_body,
        out_type=(jax.ShapeDtypeStruct((2 * (_EH + 512),), jnp.int32),
                  jax.ShapeDtypeStruct((2 * (_NP + 32),), jnp.int32)),
        mesh=mesh,
        scratch_types=[
            pltpu.VMEM((_SW,), jnp.int32),
            pltpu.VMEM((_SW,), jnp.int32),
            pltpu.VMEM((_SW,), jnp.int32),
            pltpu.VMEM((_SW,), jnp.int32),
            pltpu.VMEM((_SW,), jnp.int32),
            pltpu.VMEM((_SWT,), jnp.int32),
            pltpu.VMEM((_SWT,), jnp.int32),
            pltpu.VMEM((_SWT,), jnp.int32),
            pltpu.VMEM((_SWT,), jnp.int32),
            pltpu.VMEM((_SWT,), jnp.int32),
            pltpu.VMEM((16,), jnp.int32),
            pltpu.VMEM((16,), jnp.int32),
            pltpu.VMEM((_NPS * 16,), jnp.int32),
            pltpu.VMEM((_NPS + 32,), jnp.int32),
            pltpu.VMEM((64,), jnp.int32),
            pltpu.VMEM((1024,), jnp.int32),
            pltpu.VMEM_SHARED((802944,), jnp.int32),
            pltpu.VMEM_SHARED((1024,), jnp.int32),
            pltpu.SemaphoreType.DMA,
        ],
    )
    return fn(edge_index, zeros)


def _stats_body(ncc, b_pad, srcs, offs, s1o, s2o, mxo, mno,
                offA_v, offB_v, idxA_v, idxB_v, rowsA_v, rowsB_v,
                st1, st2, st3, st4, sem):
    wid = lax.axis_index("s") * 2 + lax.axis_index("c")
    nlo = pl.multiple_of(wid * _NPT, 8)
    pltpu.sync_copy(offs.at[pl.ds(nlo, _NPT + 24)], offA_v)
    pltpu.sync_copy(offs.at[pl.ds(nlo + (_NP + 32), _NPT + 24)], offB_v)

    def sload(ref, i):
        base = pl.multiple_of((i // 8) * 8, 8)
        v = ref[pl.ds(base, 16)]
        out = jnp.int32(0)
        for j in range(8):
            out = out + jnp.where(i - base == j, v[j], 0)
        return out

    def make_refill(h, idx_v, rows_v):
        base = h * (_EH + 512)
        def refill(wb):
            pltpu.sync_copy(
                srcs.at[pl.ds(pl.multiple_of(base + wb, 8), _W)], idx_v)
            pltpu.async_copy(b_pad.at[idx_v], rows_v, sem).wait()
        return refill

    refillA = make_refill(0, idxA_v, rowsA_v)
    refillB = make_refill(1, idxB_v, rowsB_v)

    def run_half(rows_v, refill, e_start, e_end, wb0, accs0):
        def body(e, st):
            wb = st[0]
            accs = st[1:]
            need = e >= wb + _W
            wb2 = jnp.where(need, (e // 8) * 8, wb)

            @pl.when(need)
            def _():
                refill(wb2)

            row = e - wb2
            out = []
            k = 0
            for c in range(ncc):
                v = rows_v[row, pl.ds(c * 16, 16)]
                out.append(accs[k] + v)
                out.append(accs[k + 1] + v * v)
                out.append(jnp.maximum(accs[k + 2], v))
                out.append(jnp.minimum(accs[k + 3], v))
                k += 4
            return (wb2,) + tuple(out)

        st = lax.fori_loop(e_start, e_end, body, (wb0,) + tuple(accs0))
        return st[0], st[1:]

    zero = jnp.zeros((16,), jnp.float32)
    neg = jnp.full((16,), -_BIG, jnp.float32)
    pos = jnp.full((16,), _BIG, jnp.float32)

    def node_body(i, carry):
        wbA, wbB = carry
        accs = []
        for _ in range(ncc):
            accs += [zero, zero, neg, pos]
        eA0 = sload(offA_v, i)
        eA1 = sload(offA_v, i + 1)
        eB0 = sload(offB_v, i)
        eB1 = sload(offB_v, i + 1)
        wbA2, accs = run_half(rowsA_v, refillA, eA0, eA1, wbA, accs)
        wbB2, accs = run_half(rowsB_v, refillB, eB0, eB1, wbB, accs)
        slot = lax.rem(i, _FLUSH)
        for c in range(ncc):
            st1[slot, pl.ds(c * 16, 16)] = accs[4 * c]
            st2[slot, pl.ds(c * 16, 16)] = accs[4 * c + 1]
            st3[slot, pl.ds(c * 16, 16)] = accs[4 * c + 2]
            st4[slot, pl.ds(c * 16, 16)] = accs[4 * c + 3]

        @pl.when(slot == _FLUSH - 1)
        def _():
            n0 = pl.multiple_of(nlo + i - (_FLUSH - 1), 8)
            pltpu.sync_copy(st1, s1o.at[pl.ds(n0, _FLUSH), :])
            pltpu.sync_copy(st2, s2o.at[pl.ds(n0, _FLUSH), :])
            pltpu.sync_copy(st3, mxo.at[pl.ds(n0, _FLUSH), :])
            pltpu.sync_copy(st4, mno.at[pl.ds(n0, _FLUSH), :])

        return (wbA2, wbB2)

    lax.fori_loop(0, _NPT, node_body, (jnp.int32(-2 * _W), jnp.int32(-2 * _W)))


def _sc_stats(b_pad, srcs, offs, ncc):
    mesh = plsc.VectorSubcoreMesh(core_axis_name="c", subcore_axis_name="s")
    fo = ncc * 16
    sds = jax.ShapeDtypeStruct((_NP, fo), jnp.float32)
    fn = pl.kernel(
        functools.partial(_stats_body, ncc),
        out_type=(sds, sds, sds, sds),
        mesh=mesh,
        scratch_types=[
            pltpu.VMEM((_NPT + 24,), jnp.int32),
            pltpu.VMEM((_NPT + 24,), jnp.int32),
            pltpu.VMEM((_W,), jnp.int32),
            pltpu.VMEM((_W,), jnp.int32),
            pltpu.VMEM((_W, _D), jnp.float32),
            pltpu.VMEM((_W, _D), jnp.float32),
            pltpu.VMEM((_FLUSH, fo), jnp.float32),
            pltpu.VMEM((_FLUSH, fo), jnp.float32),
            pltpu.VMEM((_FLUSH, fo), jnp.float32),
            pltpu.VMEM((_FLUSH, fo), jnp.float32),
            pltpu.SemaphoreType.DMA,
        ],
    )
    return fn(b_pad, srcs, offs)


def _tail_body(xc, w1, b1, w2, b2, w3, b3, out):
    h = jnp.maximum(xc[...] @ w1[...] + b1[...], 0.0)
    h = jnp.maximum(h @ w2[...] + b2[...], 0.0)
    out[...] = h @ w3[...] + b3[...]


def _tail(xc, p):
    return pl.pallas_call(
        _tail_body,
        out_shape=jax.ShapeDtypeStruct((_NB, 1), jnp.float32),
    )(xc, p['fc1_W'], p['fc1_b'][None, :], p['fc2_W'], p['fc2_b'][None, :],
      p['out_W'], p['out_b'][None, :])


def _pna_layer(h, srcs, offs, deg, degc, dlog, p, n):
    fi = h.shape[1]
    ncc = (fi + 15) // 16
    preW = p['c%d_pre_W' % n]
    a = h @ preW[:fi]
    b = h @ preW[fi:] + p['c%d_pre_b' % n]
    b_pad = jnp.zeros((_NP, _D), jnp.float32).at[:_NN, :fi].set(b)
    S1, S2, MX, MN = _sc_stats(b_pad, srcs, offs, ncc)
    S1 = S1[:_NN, :fi]
    S2 = S2[:_NN, :fi]
    MX = MX[:_NN, :fi]
    MN = MN[:_NN, :fi]
    degcol = deg[:, None]
    mean = (degcol * a + S1) / degc
    sq = (degcol * (a * a) + 2.0 * a * S1 + S2) / degc
    std = jnp.sqrt(jax.nn.relu(sq - mean * mean) + 1e-5)
    has = (deg > 0.0)[:, None]
    mx = jnp.where(has, a + MX, 0.0)
    mn = jnp.where(has, a + MN, 0.0)
    aggr = jnp.concatenate([mean, mx, mn, std], axis=-1)
    out = jnp.concatenate(
        [h, aggr, aggr * (dlog / _AVG_LOG), aggr * (_AVG_LOG / dlog)], axis=-1)
    out = out @ p['c%d_post_W' % n] + p['c%d_post_b' % n]
    out = out @ p['c%d_lin_W' % n] + p['c%d_lin_b' % n]
    out = (out - p['bn%d_rm' % n]) / jnp.sqrt(p['bn%d_rv' % n] + 1e-5) \
        * p['bn%d_g' % n] + p['bn%d_b' % n]
    return jax.nn.relu(out)


def kernel(x, params, edge_index, batch, target):
    p = params
    edge_pad = jnp.pad(edge_index, ((0, 0), (0, 16))).reshape(-1)
    zeros = jnp.zeros((802832,), jnp.int32)
    srcs, offs = _sc_sort(edge_pad, zeros)
    off2 = offs.reshape(2, _NP + 32)
    offd = off2[:, 1:_NN + 1] - off2[:, :_NN]
    deg = (offd[0] + offd[1]).astype(jnp.float32)
    degc = jnp.clip(deg, 1.0)[:, None]
    dlog = jnp.log(jnp.clip(deg, 1.0) + 1.0)[:, None]

    h = x
    for n in (1, 2, 3):
        h = _pna_layer(h, srcs, offs, deg, degc, dlog, p, n)

    sums = jax.ops.segment_sum(h, batch, num_segments=_NB)
    cnt = jnp.clip(jax.ops.segment_sum(jnp.ones((_NN,), jnp.float32), batch,
                                       num_segments=_NB), 1.0)[:, None]
    xg = jax.nn.relu((sums / cnt) @ p['fc1_xd_W'] + p['fc1_xd_b'])

    emb = p['emb'][target]
    et = jnp.transpose(emb, (0, 2, 1))
    conv = jax.lax.conv_general_dilated(
        et, p['cxt_W'], (1,), 'VALID',
        dimension_numbers=('NCH', 'OIH', 'NCH')) + p['cxt_b'][None, :, None]
    xt = conv.reshape(_NB, 32 * 78) @ p['fc1_xt_W'] + p['fc1_xt_b']

    xc = jnp.concatenate([xg, xt], axis=1)
    return _tail(xc, p)


# speedup vs baseline: 8.6898x; 1.1233x over previous
import functools

import jax
import jax.numpy as jnp
import numpy as np
from jax import lax
from jax.experimental import pallas as pl
from jax.experimental.pallas import tpu as pltpu
from jax.experimental.pallas import tpu_sc as plsc

_AVG_LOG = float(np.log(17.0))
_NN = 50000
_NB = 128
_NP = 50176
_NPT = 1568
_EH = 400000
_ET = 25000
_W = 384
_SW = 2048
_SWT = 432
_NPS = 3136
_FLUSH = 32
_BIG = 3.0e38
_D = 128


def _shift_down(v, s, iota):
    return v[jnp.maximum(iota - s, 0)]


def _sort_body(edge, zeros, src_out, off_out,
               win_d, win_s, idx_b, pos_b, ones_b,
               win_dt, win_st, idx_bt, pos_bt, ones_bt,
               g16, ones16, scan_buf, off_stage, tot_w, tot_v,
               hist, totals_sp, sem):
    c = lax.axis_index("c")
    t = lax.axis_index("s")
    iota = lax.iota(jnp.int32, 16)
    ebase = c * _EH + t * _ET
    srow = 0
    drow = _EH * 2 + 16

    z0 = pl.multiple_of(t * 50176, 8)
    pltpu.sync_copy(zeros.at[pl.ds(z0, 50176)], hist.at[pl.ds(z0, 50176)])

    @pl.when(t == 0)
    def _():
        pltpu.sync_copy(zeros.at[pl.ds(0, 128)],
                        hist.at[pl.ds(802816, 128)])

    def fill_ones(ref, n):
        def fb(i, carry):
            ref[pl.ds(i * 16, 16)] = jnp.ones((16,), jnp.int32)
            return carry
        lax.fori_loop(0, n // 16, fb, 0)

    fill_ones(ones_b, _SW)
    fill_ones(ones_bt, _SWT)
    ones16[pl.ds(0, 16)] = jnp.ones((16,), jnp.int32)
    plsc.subcore_barrier()

    def hist_window(base_e, nw, nvalid, wd, ib, ob):
        pltpu.sync_copy(edge.at[pl.ds(drow + base_e, nw)], wd)

        def ib_body(i, carry):
            d16 = wd[pl.ds(i * 16, 16)]
            valid = (i * 16 + iota) < nvalid
            ib[pl.ds(i * 16, 16)] = jnp.where(valid, d16 * 16 + t,
                                              802816 + t)
            return carry
        lax.fori_loop(0, nw // 16, ib_body, 0)
        pltpu.sync_copy(ob, hist.at[ib], add=True)

    for w in range(12):
        hist_window(pl.multiple_of(ebase + w * _SW, 8), _SW, _SW,
                    win_d, idx_b, ones_b)
    hist_window(pl.multiple_of(ebase + 12 * _SW, 8), _SWT, _ET - 12 * _SW,
                win_dt, idx_bt, ones_bt)
    plsc.subcore_barrier()

    nscan0 = pl.multiple_of(t * _NPS * 16, 8)
    pltpu.sync_copy(hist.at[pl.ds(nscan0, _NPS * 16)], scan_buf)

    def group_body(g, carry):
        off_acc = jnp.zeros((16,), jnp.int32)
        for j in range(16):
            b0 = pl.multiple_of((g * 16 + j) * 16, 8)
            v = scan_buf[pl.ds(b0, 16)]
            incl = v
            for s in (1, 2, 4, 8):
                incl = incl + jnp.where(iota >= s,
                                        _shift_down(incl, s, iota), 0)
            total = incl[15]
            scan_buf[pl.ds(b0, 16)] = (incl - v) + carry
            off_acc = jnp.where(iota == j, carry, off_acc)
            carry = carry + total
        off_stage[pl.ds(pl.multiple_of(g * 16, 8), 16)] = off_acc
        return carry

    range_total = lax.fori_loop(0, _NPS // 16, group_body, jnp.int32(0))
    for i in range(4):
        tot_w[pl.ds(i * 16, 16)] = jnp.zeros((16,), jnp.int32) + range_total
    pltpu.sync_copy(tot_w, totals_sp.at[pl.ds(pl.multiple_of(t * 64, 8), 64)])
    plsc.subcore_barrier()

    my_base = jnp.int32(0)
    pltpu.sync_copy(totals_sp, tot_v)
    for t2 in range(16):
        v = tot_v[pl.ds(t2 * 64, 16)]
        my_base = my_base + jnp.where(t2 < t, v[0], 0)

    def add_base(i, carry):
        b0 = pl.multiple_of(i * 16, 8)
        scan_buf[pl.ds(b0, 16)] = scan_buf[pl.ds(b0, 16)] + my_base
        return carry
    lax.fori_loop(0, _NPS, add_base, 0)

    def add_base_off(i, carry):
        b0 = pl.multiple_of(i * 16, 8)
        off_stage[pl.ds(b0, 16)] = off_stage[pl.ds(b0, 16)] + my_base
        return carry
    lax.fori_loop(0, _NPS // 16, add_base_off, 0)

    pltpu.sync_copy(scan_buf, hist.at[pl.ds(nscan0, _NPS * 16)])
    @pl.when(t == 15)
    def _():
        def pb(i, carry):
            off_stage[pl.ds(pl.multiple_of(_NPS + i * 16, 8), 16)] = (
                jnp.zeros((16,), jnp.int32) + jnp.int32(_EH))
            return carry
        lax.fori_loop(0, 2, pb, 0)

    @pl.when(t < 15)
    def _():
        pltpu.sync_copy(
            off_stage.at[pl.ds(0, _NPS)],
            off_out.at[pl.ds(pl.multiple_of(c * (_NP + 32) + t * _NPS, 8),
                             _NPS)])

    @pl.when(t == 15)
    def _():
        pltpu.sync_copy(
            off_stage,
            off_out.at[pl.ds(pl.multiple_of(c * (_NP + 32) + 15 * _NPS, 8),
                             _NPS + 32)])
    plsc.subcore_barrier()

    def perm_window(base_e, nw, nvalid, wd, ws, pb):
        pltpu.sync_copy(edge.at[pl.ds(drow + base_e, nw)], wd)
        pltpu.sync_copy(edge.at[pl.ds(srow + base_e, nw)], ws)

        def vb(i, carry):
            d16 = wd[pl.ds(i * 16, 16)]
            valid = (i * 16 + iota) < nvalid
            idx16 = jnp.where(valid, d16 * 16 + t, 802816 + t)
            pltpu.sync_copy(hist.at[idx16], g16)
            b16 = g16[pl.ds(0, 16)]
            rank = jnp.zeros((16,), jnp.int32)
            for s in range(1, 16):
                dn = _shift_down(d16, s, iota)
                rank = rank + jnp.where((iota >= s) & (dn == d16), 1, 0)
            pb[pl.ds(i * 16, 16)] = (c * (_EH + 512)
                                     + jnp.where(valid, b16 + rank,
                                                 _EH + 16 + iota))
            pltpu.sync_copy(ones16, hist.at[idx16], add=True)
            return carry
        lax.fori_loop(0, nw // 16, vb, 0)
        pltpu.sync_copy(ws, src_out.at[pb])

    for w in range(12):
        perm_window(pl.multiple_of(ebase + w * _SW, 8), _SW, _SW,
                    win_d, win_s, pos_b)
    perm_window(pl.multiple_of(ebase + 12 * _SW, 8), _SWT, _ET - 12 * _SW,
                win_dt, win_st, pos_bt)
    plsc.subcore_barrier()

    @pl.when(t == 0)
    def _():
        pltpu.sync_copy(
            ones_b.at[pl.ds(0, 512)],
            src_out.at[pl.ds(pl.multiple_of(c * (_EH + 512) + _EH, 8), 512)])


def _sc_sort(edge_index, zeros):
    mesh = plsc.VectorSubcoreMesh(core_axis_name="c", subcore_axis_name="s")
    fn = pl.kernel(
        _sort_body,
        out_type=(jax.ShapeDtypeStruct((2 * (_EH + 512),), jnp.int32),
                  jax.ShapeDtypeStruct((2 * (_NP + 32),), jnp.int32)),
        mesh=mesh,
        scratch_types=[
            pltpu.VMEM((_SW,), jnp.int32),
            pltpu.VMEM((_SW,), jnp.int32),
            pltpu.VMEM((_SW,), jnp.int32),
            pltpu.VMEM((_SW,), jnp.int32),
            pltpu.VMEM((_SW,), jnp.int32),
            pltpu.VMEM((_SWT,), jnp.int32),
            pltpu.VMEM((_SWT,), jnp.int32),
            pltpu.VMEM((_SWT,), jnp.int32),
            pltpu.VMEM((_SWT,), jnp.int32),
            pltpu.VMEM((_SWT,), jnp.int32),
            pltpu.VMEM((16,), jnp.int32),
            pltpu.VMEM((16,), jnp.int32),
            pltpu.VMEM((_NPS * 16,), jnp.int32),
            pltpu.VMEM((_NPS + 32,), jnp.int32),
            pltpu.VMEM((64,), jnp.int32),
            pltpu.VMEM((1024,), jnp.int32),
            pltpu.VMEM_SHARED((802944,), jnp.int32),
            pltpu.VMEM_SHARED((1024,), jnp.int32),
            pltpu.SemaphoreType.DMA,
        ],
    )
    return fn(edge_index, zeros)


def _stats_body(ncc, b_pad, srcs, offs, s1o, s2o, mxo, mno,
                offA_v, offB_v, idxA_v, idxB_v, rowsA_v, rowsB_v,
                st1, st2, st3, st4, sem):
    wid = lax.axis_index("s") * 2 + lax.axis_index("c")
    nlo = pl.multiple_of(wid * _NPT, 8)
    pltpu.sync_copy(offs.at[pl.ds(nlo, _NPT + 24)], offA_v)
    pltpu.sync_copy(offs.at[pl.ds(nlo + (_NP + 32), _NPT + 24)], offB_v)

    def sload(ref, i):
        base = pl.multiple_of((i // 8) * 8, 8)
        v = ref[pl.ds(base, 16)]
        out = jnp.int32(0)
        for j in range(8):
            out = out + jnp.where(i - base == j, v[j], 0)
        return out

    def make_refill(h, idx_v, rows_v):
        base = h * (_EH + 512)
        def refill(wb):
            pltpu.sync_copy(
                srcs.at[pl.ds(pl.multiple_of(base + wb, 8), _W)], idx_v)
            pltpu.async_copy(b_pad.at[idx_v], rows_v, sem).wait()
        return refill

    refillA = make_refill(0, idxA_v, rowsA_v)
    refillB = make_refill(1, idxB_v, rowsB_v)

    def run_half(rows_v, refill, e_start, e_end, wb0, accs0):
        def body(e, st):
            wb = st[0]
            accs = st[1:]
            need = e >= wb + _W
            wb2 = jnp.where(need, (e // 8) * 8, wb)

            @pl.when(need)
            def _():
                refill(wb2)

            row = e - wb2
            out = []
            k = 0
            for c in range(ncc):
                v = rows_v[row, pl.ds(c * 16, 16)]
                out.append(accs[k] + v)
                out.append(accs[k + 1] + v * v)
                out.append(jnp.maximum(accs[k + 2], v))
                out.append(jnp.minimum(accs[k + 3], v))
                k += 4
            return (wb2,) + tuple(out)

        st = lax.fori_loop(e_start, e_end, body, (wb0,) + tuple(accs0))
        return st[0], st[1:]

    zero = jnp.zeros((16,), jnp.float32)
    neg = jnp.full((16,), -_BIG, jnp.float32)
    pos = jnp.full((16,), _BIG, jnp.float32)

    def node_body(i, carry):
        wbA, wbB = carry
        accs = []
        for _ in range(ncc):
            accs += [zero, zero, neg, pos]
        eA0 = sload(offA_v, i)
        eA1 = sload(offA_v, i + 1)
        eB0 = sload(offB_v, i)
        eB1 = sload(offB_v, i + 1)
        wbA2, accs = run_half(rowsA_v, refillA, eA0, eA1, wbA, accs)
        wbB2, accs = run_half(rowsB_v, refillB, eB0, eB1, wbB, accs)
        slot = lax.rem(i, _FLUSH)
        for c in range(ncc):
            st1[slot, pl.ds(c * 16, 16)] = accs[4 * c]
            st2[slot, pl.ds(c * 16, 16)] = accs[4 * c + 1]
            st3[slot, pl.ds(c * 16, 16)] = accs[4 * c + 2]
            st4[slot, pl.ds(c * 16, 16)] = accs[4 * c + 3]

        @pl.when(slot == _FLUSH - 1)
        def _():
            n0 = pl.multiple_of(nlo + i - (_FLUSH - 1), 8)
            pltpu.sync_copy(st1, s1o.at[pl.ds(n0, _FLUSH), :])
            pltpu.sync_copy(st2, s2o.at[pl.ds(n0, _FLUSH), :])
            pltpu.sync_copy(st3, mxo.at[pl.ds(n0, _FLUSH), :])
            pltpu.sync_copy(st4, mno.at[pl.ds(n0, _FLUSH), :])

        return (wbA2, wbB2)

    lax.fori_loop(0, _NPT, node_body, (jnp.int32(-2 * _W), jnp.int32(-2 * _W)))


def _sc_stats(b_pad, srcs, offs, ncc):
    mesh = plsc.VectorSubcoreMesh(core_axis_name="c", subcore_axis_name="s")
    fo = ncc * 16
    sds = jax.ShapeDtypeStruct((_NP, fo), jnp.float32)
    fn = pl.kernel(
        functools.partial(_stats_body, ncc),
        out_type=(sds, sds, sds, sds),
        mesh=mesh,
        scratch_types=[
            pltpu.VMEM((_NPT + 24,), jnp.int32),
            pltpu.VMEM((_NPT + 24,), jnp.int32),
            pltpu.VMEM((_W,), jnp.int32),
            pltpu.VMEM((_W,), jnp.int32),
            pltpu.VMEM((_W, _D), jnp.float32),
            pltpu.VMEM((_W, _D), jnp.float32),
            pltpu.VMEM((_FLUSH, fo), jnp.float32),
            pltpu.VMEM((_FLUSH, fo), jnp.float32),
            pltpu.VMEM((_FLUSH, fo), jnp.float32),
            pltpu.VMEM((_FLUSH, fo), jnp.float32),
            pltpu.SemaphoreType.DMA,
        ],
    )
    return fn(b_pad, srcs, offs)


_NBLK = 400


def _pre_body(fi, h_ref, w_ref, b_ref, a_ref, bp_ref):
    h = h_ref[...]
    w = w_ref[...]
    a_ref[...] = jax.lax.dot_general(h, w[:fi], (((1,), (0,)), ((), ())),
                                     preferred_element_type=jnp.float32)
    b = jax.lax.dot_general(h, w[fi:], (((1,), (0,)), ((), ())),
                            preferred_element_type=jnp.float32) + b_ref[...]
    bp_ref[...] = jnp.concatenate(
        [b, jnp.zeros((h.shape[0], _D - fi), jnp.float32)], axis=1)


def _tc_pre(h, preW, pre_b):
    fi = h.shape[1]
    grid = _NN // _NBLK
    return pl.pallas_call(
        functools.partial(_pre_body, fi),
        grid=(grid,),
        in_specs=[
            pl.BlockSpec((_NBLK, fi), lambda i: (i, 0)),
            pl.BlockSpec((2 * fi, fi), lambda i: (0, 0)),
            pl.BlockSpec((1, fi), lambda i: (0, 0)),
        ],
        out_specs=[
            pl.BlockSpec((_NBLK, fi), lambda i: (i, 0)),
            pl.BlockSpec((_NBLK, _D), lambda i: (i, 0)),
        ],
        out_shape=[
            jax.ShapeDtypeStruct((_NN, fi), jnp.float32),
            jax.ShapeDtypeStruct((_NP, _D), jnp.float32),
        ],
    )(h, preW, pre_b[None, :])


def _post_body(fi, h_ref, a_ref, s1_ref, s2_ref, mx_ref, mn_ref, deg_ref,
               pw_ref, pb_ref, lw_ref, lb_ref, g_ref, be_ref, rm_ref, rv_ref,
               o_ref):
    h = h_ref[...]
    a = a_ref[...]
    S1 = s1_ref[:, :fi]
    S2 = s2_ref[:, :fi]
    MX = mx_ref[:, :fi]
    MN = mn_ref[:, :fi]
    deg = deg_ref[...]
    degc = jnp.maximum(deg, 1.0)
    dlog = jnp.log(degc + 1.0)
    mean = (deg * a + S1) / degc
    sq = (deg * (a * a) + 2.0 * a * S1 + S2) / degc
    std = jnp.sqrt(jnp.maximum(sq - mean * mean, 0.0) + 1e-5)
    has = deg > 0.0
    mx = jnp.where(has, a + MX, 0.0)
    mn = jnp.where(has, a + MN, 0.0)
    aggr = jnp.concatenate([mean, mx, mn, std], axis=1)
    x13 = jnp.concatenate(
        [h, aggr, aggr * (dlog / _AVG_LOG), aggr * (_AVG_LOG / dlog)], axis=1)
    o = jax.lax.dot_general(x13, pw_ref[...], (((1,), (0,)), ((), ())),
                            preferred_element_type=jnp.float32) + pb_ref[...]
    o = jax.lax.dot_general(o, lw_ref[...], (((1,), (0,)), ((), ())),
                            preferred_element_type=jnp.float32) + lb_ref[...]
    o = (o - rm_ref[...]) / jnp.sqrt(rv_ref[...] + 1e-5) * g_ref[...] \
        + be_ref[...]
    o_ref[...] = jnp.maximum(o, 0.0)


def _tc_post(h, a, S1, S2, MX, MN, deg, p, n):
    fi = h.shape[1]
    fo = p['c%d_post_W' % n].shape[1]
    fs = S1.shape[1]
    grid = _NN // _NBLK
    vec = lambda name: p[name][None, :]
    return pl.pallas_call(
        functools.partial(_post_body, fi),
        grid=(grid,),
        in_specs=[
            pl.BlockSpec((_NBLK, fi), lambda i: (i, 0)),
            pl.BlockSpec((_NBLK, fi), lambda i: (i, 0)),
            pl.BlockSpec((_NBLK, fs), lambda i: (i, 0)),
            pl.BlockSpec((_NBLK, fs), lambda i: (i, 0)),
            pl.BlockSpec((_NBLK, fs), lambda i: (i, 0)),
            pl.BlockSpec((_NBLK, fs), lambda i: (i, 0)),
            pl.BlockSpec((_NBLK, 1), lambda i: (i, 0)),
            pl.BlockSpec((13 * fi, fo), lambda i: (0, 0)),
            pl.BlockSpec((1, fo), lambda i: (0, 0)),
            pl.BlockSpec((fo, fo), lambda i: (0, 0)),
            pl.BlockSpec((1, fo), lambda i: (0, 0)),
            pl.BlockSpec((1, fo), lambda i: (0, 0)),
            pl.BlockSpec((1, fo), lambda i: (0, 0)),
            pl.BlockSpec((1, fo), lambda i: (0, 0)),
            pl.BlockSpec((1, fo), lambda i: (0, 0)),
        ],
        out_specs=pl.BlockSpec((_NBLK, fo), lambda i: (i, 0)),
        out_shape=jax.ShapeDtypeStruct((_NN, fo), jnp.float32),
    )(h, a, S1[:_NN], S2[:_NN], MX[:_NN], MN[:_NN], deg[:, None],
      p['c%d_post_W' % n], vec('c%d_post_b' % n),
      p['c%d_lin_W' % n], vec('c%d_lin_b' % n),
      vec('bn%d_g' % n), vec('bn%d_b' % n),
      vec('bn%d_rm' % n), vec('bn%d_rv' % n))


def _pool_body(h_ref, bt_ref, o_ref):
    i = pl.program_id(0)

    @pl.when(i == 0)
    def _():
        o_ref[...] = jnp.zeros_like(o_ref)

    h = h_ref[...]
    hext = jnp.concatenate([h, jnp.ones((h.shape[0], 1), jnp.float32)],
                           axis=1)
    bt = bt_ref[0]
    onehotT = (jax.lax.broadcasted_iota(jnp.int32, (_NB, _NBLK), 0)
               == bt).astype(jnp.float32)
    o_ref[...] += jax.lax.dot_general(
        onehotT, hext, (((1,), (0,)), ((), ())),
        preferred_element_type=jnp.float32)


def _tc_pool(h, batch):
    grid = _NN // _NBLK
    b2 = batch.reshape(grid, 1, _NBLK)
    return pl.pallas_call(
        _pool_body,
        grid=(grid,),
        in_specs=[
            pl.BlockSpec((_NBLK, 64), lambda i: (i, 0)),
            pl.BlockSpec((1, 1, _NBLK), lambda i: (i, 0, 0)),
        ],
        out_specs=pl.BlockSpec((_NB, 65), lambda i: (0, 0)),
        out_shape=jax.ShapeDtypeStruct((_NB, 65), jnp.float32),
        compiler_params=pltpu.CompilerParams(
            dimension_semantics=("arbitrary",)),
    )(h, b2)


def _xt_body(tg_ref, emb_ref, wk_ref, wp_ref, bx_ref, o_ref):
    for j in range(8):
        oh_t = (jax.lax.broadcasted_iota(jnp.int32, (28, 85), 0)
                == tg_ref[0, j]).astype(jnp.float32)
        embb = jax.lax.dot_general(oh_t, emb_ref[...],
                                   (((0,), (0,)), ((), ())),
                                   preferred_element_type=jnp.float32)
        acc = jnp.zeros((78, 32), jnp.float32)
        for k in range(8):
            acc = acc + jax.lax.dot_general(
                embb[k:k + 78], wk_ref[k], (((1,), (0,)), ((), ())),
                preferred_element_type=jnp.float32)
        tmp = jnp.zeros((78, 128), jnp.float32)
        for cc in range(32):
            tmp = tmp + acc[:, cc:cc + 1] * wp_ref[:, cc, :]
        o_ref[pl.ds(j, 1), :] = (jnp.sum(tmp, axis=0)[None, :]
                                 + bx_ref[...])


def _tc_xt(target, p):
    wk = jnp.transpose(p['cxt_W'], (2, 1, 0))
    wp = p['fc1_xt_W'].reshape(32, 78, 128).transpose(1, 0, 2)
    return pl.pallas_call(
        _xt_body,
        grid=(_NB // 8,),
        in_specs=[
            pl.BlockSpec((1, 8, 85), lambda i: (i, 0, 0)),
            pl.BlockSpec((28, 128), lambda i: (0, 0)),
            pl.BlockSpec((8, 128, 32), lambda i: (0, 0, 0)),
            pl.BlockSpec((78, 32, 128), lambda i: (0, 0, 0)),
            pl.BlockSpec((1, 128), lambda i: (0, 0)),
        ],
        out_specs=pl.BlockSpec((8, 128), lambda i: (i, 0)),
        out_shape=jax.ShapeDtypeStruct((_NB, 128), jnp.float32),
    )(target.reshape(_NB // 8, 8, 85), p['emb'], wk, wp,
      _xt_bias(p)[None, :])


def _xt_bias(p):
    cb = jnp.repeat(p['cxt_b'], 78)
    return p['fc1_xt_b'] + cb @ p['fc1_xt_W']


def _tail2_body(pool_ref, xt_ref, wg_ref, bg_ref, w1, b1, w2, b2, w3, b3,
                out_ref):
    pooled = pool_ref[...]
    sums = pooled[:, :64]
    cnt = jnp.maximum(pooled[:, 64:65], 1.0)
    xg = jnp.maximum(
        jax.lax.dot_general(sums / cnt, wg_ref[...], (((1,), (0,)), ((), ())),
                            preferred_element_type=jnp.float32) + bg_ref[...],
        0.0)
    xc = jnp.concatenate([xg, xt_ref[...]], axis=1)
    h = jnp.maximum(xc @ w1[...] + b1[...], 0.0)
    h = jnp.maximum(h @ w2[...] + b2[...], 0.0)
    out_ref[...] = h @ w3[...] + b3[...]


def _tc_tail2(pooled, xt, p):
    return pl.pallas_call(
        _tail2_body,
        out_shape=jax.ShapeDtypeStruct((_NB, 1), jnp.float32),
    )(pooled, xt, p['fc1_xd_W'], p['fc1_xd_b'][None, :],
      p['fc1_W'], p['fc1_b'][None, :], p['fc2_W'], p['fc2_b'][None, :],
      p['out_W'], p['out_b'][None, :])


def _tail_body(xc, w1, b1, w2, b2, w3, b3, out):
    h = jnp.maximum(xc[...] @ w1[...] + b1[...], 0.0)
    h = jnp.maximum(h @ w2[...] + b2[...], 0.0)
    out[...] = h @ w3[...] + b3[...]


def _tail(xc, p):
    return pl.pallas_call(
        _tail_body,
        out_shape=jax.ShapeDtypeStruct((_NB, 1), jnp.float32),
    )(xc, p['fc1_W'], p['fc1_b'][None, :], p['fc2_W'], p['fc2_b'][None, :],
      p['out_W'], p['out_b'][None, :])


def _pna_layer(h, srcs, offs, deg, p, n):
    fi = h.shape[1]
    ncc = (fi + 15) // 16
    a, b_pad = _tc_pre(h, p['c%d_pre_W' % n], p['c%d_pre_b' % n])
    S1, S2, MX, MN = _sc_stats(b_pad, srcs, offs, ncc)
    return _tc_post(h, a, S1, S2, MX, MN, deg, p, n)


def kernel(x, params, edge_index, batch, target):
    p = params
    edge_pad = jnp.pad(edge_index, ((0, 0), (0, 16))).reshape(-1)
    zeros = jnp.zeros((802832,), jnp.int32)
    srcs, offs = _sc_sort(edge_pad, zeros)
    off2 = offs.reshape(2, _NP + 32)
    offd = off2[:, 1:_NN + 1] - off2[:, :_NN]
    deg = (offd[0] + offd[1]).astype(jnp.float32)
    degc = jnp.clip(deg, 1.0)[:, None]
    dlog = jnp.log(jnp.clip(deg, 1.0) + 1.0)[:, None]

    h = x
    for n in (1, 2, 3):
        h = _pna_layer(h, srcs, offs, deg, p, n)

    pooled = _tc_pool(h, batch)
    xt = _tc_xt(target, p)
    return _tc_tail2(pooled, xt, p)
